# Initial kernel scaffold; baseline (speedup 1.0000x reference)
import jax
import jax.numpy as jnp
from jax.experimental import pallas as pl


def kernel(idx_words, embed_table, W, b):
    raise NotImplementedError("write your pallas kernel here")


# speedup vs baseline: 11.2064x; 11.2064x over previous
import functools

import jax
import jax.numpy as jnp
from jax import lax
from jax.experimental import pallas as pl
from jax.experimental.pallas import tpu as pltpu
from jax.experimental.pallas import tpu_sc as plsc

BATCH = 4096
SEQ = 200
VOCAB = 100000
D_EMBED = 128
D_OUT = 64

NUM_CORES = 2
NUM_SUBCORES = 16
NUM_WORKERS = NUM_CORES * NUM_SUBCORES
ROWS_PER_W = BATCH // NUM_WORKERS
CHUNKS = 2
CHUNK = SEQ // CHUNKS
LANES = 16
COL_CHUNKS = D_OUT // LANES

BV = 2000


def _proj_body(tab_ref, wt_ref, out_ref):
    i = pl.program_id(0)
    x = tab_ref[...]
    row_ids = lax.broadcasted_iota(jnp.int32, x.shape, 0) + i * BV
    x = jnp.where(row_ids == 0, jnp.float32(0.0), x)
    out_ref[...] = lax.dot_general(
        x, wt_ref[...], (((1,), (0,)), ((), ())),
        preferred_element_type=jnp.float32)


def _project(table, wt):
    return pl.pallas_call(
        _proj_body,
        grid=(VOCAB // BV,),
        in_specs=[
            pl.BlockSpec((BV, D_EMBED), lambda i: (i, 0)),
            pl.BlockSpec((D_EMBED, D_OUT), lambda i: (0, 0)),
        ],
        out_specs=pl.BlockSpec((BV, D_OUT), lambda i: (i, 0)),
        out_shape=jax.ShapeDtypeStruct((VOCAB, D_OUT), jnp.float32),
    )(table, wt)


def _bag_body(p_hbm, idx_hbm, b_hbm, out_hbm,
              idx_v, rows_v, out_v, b_v, sem0, sem1):
    wid = lax.axis_index("s") * NUM_CORES + lax.axis_index("c")
    base = wid * ROWS_PER_W
    pltpu.sync_copy(idx_hbm.at[pl.ds(base, ROWS_PER_W)], idx_v)
    pltpu.sync_copy(b_hbm, b_v)
    bias = [b_v[pl.ds(LANES * k, LANES)] for k in range(COL_CHUNKS)]
    inv_seq = jnp.float32(1.0 / SEQ)
    sems = (sem0, sem1)

    def issue(r, buf):
        for c in range(CHUNKS):
            pltpu.async_copy(
                p_hbm.at[idx_v.at[r, c]],
                rows_v.at[buf, pl.ds(c * CHUNK, CHUNK), :],
                sems[buf])

    def drain(buf):
        pltpu.make_async_copy(
            p_hbm.at[pl.ds(0, SEQ)], rows_v.at[buf], sems[buf]).wait()

    def reduce_into(r_out, buf):
        def body(j, acc):
            return tuple(
                acc[k] + rows_v[buf, j, pl.ds(LANES * k, LANES)]
                for k in range(COL_CHUNKS))
        zero = jnp.zeros((LANES,), jnp.float32)
        acc = lax.fori_loop(0, SEQ, body, (zero,) * COL_CHUNKS, unroll=8)
        for k in range(COL_CHUNKS):
            out_v[r_out, pl.ds(LANES * k, LANES)] = acc[k] * inv_seq + bias[k]

    issue(0, 0)

    @pl.loop(0, ROWS_PER_W, step=2)
    def _(r):
        issue(r + 1, 1)
        drain(0)
        reduce_into(r, 0)

        @pl.when(r + 2 < ROWS_PER_W)
        def _():
            issue(r + 2, 0)

        drain(1)
        reduce_into(r + 1, 1)

    pltpu.sync_copy(out_v, out_hbm.at[pl.ds(base, ROWS_PER_W)])


@functools.partial(
    pl.kernel,
    out_type=jax.ShapeDtypeStruct((BATCH, D_OUT), jnp.float32),
    mesh=plsc.VectorSubcoreMesh(core_axis_name="c", subcore_axis_name="s"),
    compiler_params=pltpu.CompilerParams(use_tc_tiling_on_sc=False),
    scratch_types=[
        pltpu.VMEM((ROWS_PER_W, CHUNKS, CHUNK), jnp.int32),
        pltpu.VMEM((2, SEQ, D_OUT), jnp.float32),
        pltpu.VMEM((ROWS_PER_W, D_OUT), jnp.float32),
        pltpu.VMEM((D_OUT,), jnp.float32),
        pltpu.SemaphoreType.DMA,
        pltpu.SemaphoreType.DMA,
    ],
)
def _bag(p_hbm, idx_hbm, b_hbm, out_hbm,
         idx_v, rows_v, out_v, b_v, sem0, sem1):
    _bag_body(p_hbm, idx_hbm, b_hbm, out_hbm,
              idx_v, rows_v, out_v, b_v, sem0, sem1)


@jax.jit
def kernel(idx_words, embed_table, W, b):
    proj = _project(embed_table, W.T)
    idx3 = idx_words.reshape(BATCH, CHUNKS, CHUNK).astype(jnp.int32)
    return _bag(proj, idx3, b)
